# Initial kernel scaffold; baseline (speedup 1.0000x reference)
import jax
import jax.numpy as jnp
from jax.experimental import pallas as pl


def kernel(points, vertices, faces, mask):
    raise NotImplementedError("write your pallas kernel here")


# speedup vs baseline: 5.0880x; 5.0880x over previous
import functools

import jax
import jax.numpy as jnp
from jax import lax
from jax.experimental import pallas as pl
from jax.experimental.pallas import tpu as pltpu
from jax.experimental.pallas import tpu_sc as plsc

NC = 2
NS = 16
NW = NC * NS
L = 16

PF = 3200
NI = 3 * PF
GB = 128
NG = NI // GB
FP = NW * PF
CF = 128
CR = 3 * CF
NCH = PF // CF
VD = 16


def _make_kernel():
    mesh = plsc.VectorSubcoreMesh(
        core_axis_name="c", subcore_axis_name="s", num_cores=NC,
        num_subcores=NS)

    @functools.partial(
        pl.kernel,
        out_type=jax.ShapeDtypeStruct((NW, 2, L), jnp.float32),
        mesh=mesh,
        compiler_params=pltpu.CompilerParams(
            needs_layout_passes=False, use_tc_tiling_on_sc=False),
        scratch_types=[
            pltpu.VMEM((NG, GB), jnp.int32),
            pltpu.VMEM((CR, VD), jnp.float32),
            pltpu.VMEM((CR, VD), jnp.float32),
            pltpu.VMEM((PF * 3,), jnp.float32),
            pltpu.VMEM((PF,), jnp.float32),
            pltpu.VMEM((2, L), jnp.float32),
            pltpu.SemaphoreType.DMA,
            pltpu.SemaphoreType.DMA,
        ],
    )
    def beam_gap(ff_hbm, pts_hbm, mf_hbm, vert_hbm, out_hbm,
                 idx_v, rows_a, rows_b, pts_v, mf_v, out_v, sem_a, sem_b):
        wid = lax.axis_index("s") * NC + lax.axis_index("c")
        pltpu.sync_copy(ff_hbm.at[wid], idx_v)
        pltpu.sync_copy(pts_hbm.at[wid], pts_v)
        pltpu.sync_copy(mf_hbm.at[wid], mf_v)

        bufs = (rows_a, rows_b)
        sems = (sem_a, sem_b)

        def fire(c):
            buf, sem = bufs[c % 2], sems[c % 2]
            return [
                pltpu.async_copy(
                    vert_hbm.at[idx_v.at[3 * c + g]],
                    buf.at[pl.ds(g * GB, GB), :], sem)
                for g in range(3)
            ]

        iota = lax.iota(jnp.int32, L)
        k0 = jnp.zeros((L,), jnp.int32)
        k1 = jnp.full((L,), 1, jnp.int32)
        k2 = jnp.full((L,), 2, jnp.int32)
        third = jnp.float32(1.0 / 3.0)

        def compute(c, carry):
            buf = bufs[c % 2]

            def body(i, carry):
                acc, cnt = carry
                t = i * L + iota
                b = t * 3
                ax = plsc.load_gather(buf, [b, k0])
                ay = plsc.load_gather(buf, [b, k1])
                az = plsc.load_gather(buf, [b, k2])
                bx = plsc.load_gather(buf, [b + 1, k0])
                by = plsc.load_gather(buf, [b + 1, k1])
                bz = plsc.load_gather(buf, [b + 1, k2])
                cx = plsc.load_gather(buf, [b + 2, k0])
                cy = plsc.load_gather(buf, [b + 2, k1])
                cz = plsc.load_gather(buf, [b + 2, k2])
                p = (c * CF + t) * 3
                px = plsc.load_gather(pts_v, [p])
                py = plsc.load_gather(pts_v, [p + 1])
                pz = plsc.load_gather(pts_v, [p + 2])
                m = mf_v[pl.ds(c * CF + i * L, L)]
                dx = px - (ax + bx + cx) * third
                dy = py - (ay + by + cy) * third
                dz = pz - (az + bz + cz) * third
                s = dx * dx + dy * dy + dz * dz
                sc = jnp.maximum(s, jnp.float32(1e-30))
                half_sc = sc * jnp.float32(0.5)
                yi = jnp.int32(0x5F3759DF) - (plsc.bitcast(sc, jnp.int32) >> 1)
                y = plsc.bitcast(yi, jnp.float32)
                y = y * (jnp.float32(1.5) - half_sc * y * y)
                y = y * (jnp.float32(1.5) - half_sc * y * y)
                y = y * (jnp.float32(1.5) - half_sc * y * y)
                norm = s * y
                return acc + norm * m, cnt + m

            return lax.fori_loop(0, CF // L, body, carry)

        zero = jnp.zeros((L,), jnp.float32)
        carry = (zero, zero)
        inflight = fire(0)
        for c in range(1, NCH):
            next_inflight = fire(c)
            for cp in inflight:
                cp.wait()
            carry = compute(c - 1, carry)
            inflight = next_inflight
        for cp in inflight:
            cp.wait()
        acc, cnt = compute(NCH - 1, carry)

        out_v[0, :] = acc
        out_v[1, :] = cnt
        pltpu.sync_copy(out_v, out_hbm.at[wid])

    return beam_gap


_beam_gap = _make_kernel()


def kernel(points, vertices, faces, mask):
    f = points.shape[0]
    ff = faces.reshape(-1).astype(jnp.int32)
    ff = jnp.pad(ff, (0, 3 * FP - ff.shape[0])).reshape(NW, NG, GB)
    pts = jnp.pad(points, ((0, FP - f), (0, 0))).reshape(NW, PF * 3)
    mf = jnp.pad(mask.astype(jnp.float32), (0, FP - f)).reshape(NW, PF)
    vp = jnp.pad(vertices, ((0, 0), (0, VD - 3)))
    out = _beam_gap(ff, pts, mf, vp)
    total = jnp.sum(out[:, 0, :])
    count = jnp.sum(out[:, 1, :])
    return (10.0 * total / jnp.maximum(count, 1.0)).astype(jnp.float32)
